# Initial kernel scaffold; baseline (speedup 1.0000x reference)
import jax
import jax.numpy as jnp
from jax.experimental import pallas as pl


def kernel(x, W, b):
    raise NotImplementedError("write your pallas kernel here")


# speedup vs baseline: 2.2025x; 2.2025x over previous
import functools

import jax
import jax.numpy as jnp
from jax import lax
from jax.experimental import pallas as pl
from jax.experimental.pallas import tpu as pltpu
from jax.experimental.pallas import tpu_sc as plsc

HIDDEN = 768
EXPERTS = 64
TOKENS = 4 * 8192
TC_BLK = 1024
TPW = TOKENS // 32
GRP = 16


def _tc_body(x_ref, w_ref, b_ref, lt_ref, s_ref):
    xb = x_ref[...]
    w = w_ref[...]
    lg = lax.dot_general(w, xb, (((1,), (1,)), ((), ())),
                         preferred_element_type=jnp.float32)
    lg = lg + b_ref[...]
    m = jnp.max(lg, axis=0, keepdims=True)
    s = jnp.sum(jnp.exp(lg - m), axis=0, keepdims=True)
    lt_ref[...] = lg
    s_ref[...] = s


_tc_project = pl.pallas_call(
    _tc_body,
    grid=(TOKENS // TC_BLK,),
    in_specs=[
        pl.BlockSpec((TC_BLK, HIDDEN), lambda i: (i, 0)),
        pl.BlockSpec((EXPERTS, HIDDEN), lambda i: (0, 0)),
        pl.BlockSpec((EXPERTS, 1), lambda i: (0, 0)),
    ],
    out_specs=[
        pl.BlockSpec((EXPERTS, TC_BLK), lambda i: (0, i)),
        pl.BlockSpec((1, TC_BLK), lambda i: (0, i)),
    ],
    out_shape=[
        jax.ShapeDtypeStruct((EXPERTS, TOKENS), jnp.float32),
        jax.ShapeDtypeStruct((1, TOKENS), jnp.float32),
    ],
    compiler_params=pltpu.CompilerParams(
        dimension_semantics=("arbitrary",)),
)


@functools.partial(
    pl.kernel,
    mesh=plsc.VectorSubcoreMesh(core_axis_name="c", subcore_axis_name="s"),
    out_type=[
        jax.ShapeDtypeStruct((2, TOKENS), jnp.float32),
        jax.ShapeDtypeStruct((2, TOKENS), jnp.int32),
    ],
    scratch_types=[
        pltpu.VMEM((EXPERTS, TPW), jnp.float32),
        pltpu.VMEM((1, TPW), jnp.float32),
        pltpu.VMEM((2, TPW), jnp.float32),
        pltpu.VMEM((2, TPW), jnp.int32),
    ],
)
def _sc_top2(lt_hbm, s_hbm, g_hbm, i_hbm, lt_v, s_v, g_v, i_v):
    wid = lax.axis_index("s") * 2 + lax.axis_index("c")
    base = wid * TPW
    pltpu.sync_copy(lt_hbm.at[:, pl.ds(base, TPW)], lt_v)
    pltpu.sync_copy(s_hbm.at[:, pl.ds(base, TPW)], s_v)

    def group(g, carry):
        ts = g * GRP
        m1 = jnp.full((GRP,), -jnp.inf, jnp.float32)
        m2 = m1
        i1 = jnp.zeros((GRP,), jnp.int32)
        i2 = i1
        for e in range(EXPERTS):
            v = lt_v[e, pl.ds(ts, GRP)]
            gt1 = v > m1
            gt2 = v > m2
            m2 = jnp.where(gt1, m1, jnp.where(gt2, v, m2))
            i2 = jnp.where(gt1, i1, jnp.where(gt2, e, i2))
            m1 = jnp.where(gt1, v, m1)
            i1 = jnp.where(gt1, e, i1)
        inv = 1.0 / s_v[0, pl.ds(ts, GRP)]
        g2 = jnp.exp(m2 - m1) * inv
        g_v[0, pl.ds(ts, GRP)] = inv
        g_v[1, pl.ds(ts, GRP)] = g2
        i_v[0, pl.ds(ts, GRP)] = i1
        i_v[1, pl.ds(ts, GRP)] = i2
        return carry

    lax.fori_loop(0, TPW // GRP, group, 0)
    pltpu.sync_copy(g_v, g_hbm.at[:, pl.ds(base, TPW)])
    pltpu.sync_copy(i_v, i_hbm.at[:, pl.ds(base, TPW)])


def kernel(x, W, b):
    xf = x.reshape(TOKENS, HIDDEN)
    lt, s = _tc_project(xf, W, b.reshape(EXPERTS, 1))
    g, i = _sc_top2(lt, s)
    bsz, seq = x.shape[0], x.shape[1]
    return g.T.reshape(bsz, seq, 2), i.T.reshape(bsz, seq, 2)
